# baseline (speedup 1.0000x reference)
import jax
import jax.numpy as jnp
from jax import lax
from jax.experimental import pallas as pl
from jax.experimental.pallas import tpu as pltpu
from jax.experimental.pallas import tpu_sc as plsc

_SEQ = 4096
_HID = 4096
_SLOTS = 64
_K = 8
_LANES = 16
_NBLK = 8
_ROWS = _SEQ // _NBLK


def _norms_topk_body(x_ref, k_ref, i_ref):
    x = x_ref[...]
    n = jnp.sum(x * x, axis=1)[None, :]
    blk = pl.program_id(0)
    iota = lax.broadcasted_iota(jnp.int32, (1, _ROWS), 1)
    lane8 = lax.broadcasted_iota(jnp.int32, (1, _K), 1)
    kvec = jnp.full((1, _K), -jnp.inf, jnp.float32)
    ivec = jnp.zeros((1, _K), jnp.int32)
    for t in range(_K):
        m = jnp.max(n)
        i = jnp.min(jnp.where(n == m, iota, jnp.int32(2**30)))
        kvec = jnp.where(lane8 == t, m, kvec)
        ivec = jnp.where(lane8 == t, i + blk * _ROWS, ivec)
        n = jnp.where(iota == i, -jnp.inf, n)
    k_ref[...] = kvec[None]
    i_ref[...] = ivec[None]


def _tc_norms_topk(pred2d):
    return pl.pallas_call(
        _norms_topk_body,
        grid=(_NBLK,),
        in_specs=[pl.BlockSpec((_ROWS, _HID), lambda i: (i, 0))],
        out_specs=[
            pl.BlockSpec((1, 1, _K), lambda i: (i, 0, 0)),
            pl.BlockSpec((1, 1, _K), lambda i: (i, 0, 0)),
        ],
        out_shape=[
            jax.ShapeDtypeStruct((_NBLK, 1, _K), jnp.float32),
            jax.ShapeDtypeStruct((_NBLK, 1, _K), jnp.int32),
        ],
    )(pred2d)


def _merge_sorted(bk, bi, ck_s, ci_s):
    ck_r = lax.rev(ck_s, (0,))
    ci_r = lax.rev(ci_s, (0,))
    keep = bk >= ck_r
    mk = jnp.where(keep, bk, ck_r)
    mi = jnp.where(keep, bi, ci_r)
    nk, ni = plsc.sort_key_val(mk, mi, descending=True)
    return nk, ni


def _sc_body(keys_hbm, idxs_hbm, pred_hbm, states_hbm, strength_hbm,
             out_states_hbm, out_strength_hbm,
             kf, ivf, idx_v, rows_v, str_v, bank_v, sem):
    c = lax.axis_index("c")
    s = lax.axis_index("s")

    @pl.when((c == 0) & (s == 0))
    def _topk():
        pltpu.sync_copy(keys_hbm, kf)
        pltpu.sync_copy(idxs_hbm, ivf)

        def final_merge(j, carry):
            bk2, bi2 = carry
            ck = kf[pl.ds(j * _LANES, _LANES)]
            ci = ivf[pl.ds(j * _LANES, _LANES)]
            ck_s, ci_s = plsc.sort_key_val(ck, ci, descending=True)
            return _merge_sorted(bk2, bi2, ck_s, ci_s)

        bk0 = jnp.full((_LANES,), -jnp.inf, jnp.float32)
        bi0 = jnp.zeros((_LANES,), jnp.int32)
        fk, fi = lax.fori_loop(0, _NBLK * _K // _LANES, final_merge,
                               (bk0, bi0))
        idx_v[...] = fi
        pltpu.async_copy(pred_hbm.at[idx_v.at[pl.ds(0, _K)]], rows_v,
                         sem).wait()
        pltpu.sync_copy(rows_v, out_states_hbm.at[pl.ds(0, _K)])

    @pl.when((c == 1) & (s < 14))
    def _copy_bank():
        r0 = _K + s * 4
        pltpu.sync_copy(states_hbm.at[pl.ds(r0, 4)], bank_v)
        pltpu.sync_copy(bank_v, out_states_hbm.at[pl.ds(r0, 4)])

    @pl.when((c == 1) & (s == 14))
    def _strength():
        lane = lax.iota(jnp.int32, _LANES)
        pltpu.sync_copy(strength_hbm, str_v)
        s0 = str_v[pl.ds(0, _LANES)]
        str_v[pl.ds(0, _LANES)] = jnp.where(lane < _K, jnp.float32(1.0), s0)
        pltpu.sync_copy(str_v, out_strength_hbm)


def kernel(predictions, mem_states, mem_strength, top_k):
    del top_k
    pred2d = predictions.reshape(2 * _SEQ, _HID)
    keys, idxs = _tc_norms_topk(pred2d)
    keys = keys.reshape(_NBLK * _K)
    idxs = idxs.reshape(_NBLK * _K)
    sc = pl.kernel(
        _sc_body,
        mesh=plsc.VectorSubcoreMesh(core_axis_name="c", subcore_axis_name="s"),
        compiler_params=pltpu.CompilerParams(needs_layout_passes=False),
        out_type=[
            jax.ShapeDtypeStruct((_SLOTS, _HID), jnp.float32),
            jax.ShapeDtypeStruct((_SLOTS,), jnp.float32),
        ],
        scratch_types=[
            pltpu.VMEM((_NBLK * _K,), jnp.float32),
            pltpu.VMEM((_NBLK * _K,), jnp.int32),
            pltpu.VMEM((_LANES,), jnp.int32),
            pltpu.VMEM((_K, _HID), jnp.float32),
            pltpu.VMEM((_SLOTS,), jnp.float32),
            pltpu.VMEM((4, _HID), jnp.float32),
            pltpu.SemaphoreType.DMA,
        ],
    )
    new_states, new_strength = sc(keys, idxs, pred2d, mem_states, mem_strength)
    return new_states, new_strength


# speedup vs baseline: 1.2274x; 1.2274x over previous
import jax
import jax.numpy as jnp
from jax import lax
from jax.experimental import pallas as pl
from jax.experimental.pallas import tpu as pltpu
from jax.experimental.pallas import tpu_sc as plsc

_SEQ = 4096
_HID = 4096
_SLOTS = 64
_K = 8
_LANES = 16
_NTILES = 16
_PER_TILE = _SEQ // _NTILES
_NCHUNK = _PER_TILE // _LANES
_PACK = 2 * _LANES


def _norms_body(x_ref, o_ref):
    x = x_ref[...]
    o_ref[...] = jnp.sum(x * x, axis=1, keepdims=True)[None]


def _tc_norms(pred2d):
    nblk = 8
    rows = _SEQ // nblk
    return pl.pallas_call(
        _norms_body,
        grid=(nblk,),
        in_specs=[pl.BlockSpec((rows, _HID), lambda i: (i, 0))],
        out_specs=pl.BlockSpec((1, rows, 1), lambda i: (i, 0, 0)),
        out_shape=jax.ShapeDtypeStruct((nblk, rows, 1), jnp.float32),
    )(pred2d)


def _merge_sorted(bk, bi, ck_s, ci_s):
    ck_r = lax.rev(ck_s, (0,))
    ci_r = lax.rev(ci_s, (0,))
    keep = bk >= ck_r
    mk = jnp.where(keep, bk, ck_r)
    mi = jnp.where(keep, bi, ci_r)
    nk, ni = plsc.sort_key_val(mk, mi, descending=True)
    return nk, ni


def _sc_body(norms_hbm, pred_hbm, states_hbm, strength_hbm,
             out_states_hbm, out_strength_hbm,
             norms_v, pack_v, packf, idx_v, rows_v, str_v, bank_v,
             sh_p, sem):
    c = lax.axis_index("c")
    s = lax.axis_index("s")

    @pl.when(c == 0)
    def _topk():
        base = pl.multiple_of(s * _PER_TILE, _PER_TILE)
        pltpu.sync_copy(norms_hbm.at[pl.ds(base, _PER_TILE)], norms_v)
        lane = lax.iota(jnp.int32, _LANES)

        def local_merge(j, carry):
            bk, bi = carry
            ck = norms_v[pl.ds(j * _LANES, _LANES)]
            ci = lane + (base + j * _LANES)
            ck_s, ci_s = plsc.sort_key_val(ck, ci, descending=True)
            return _merge_sorted(bk, bi, ck_s, ci_s)

        bk0 = jnp.full((_LANES,), -jnp.inf, jnp.float32)
        bi0 = jnp.zeros((_LANES,), jnp.int32)
        bk, bi = lax.fori_loop(0, _NCHUNK, local_merge, (bk0, bi0))
        pack_v[pl.ds(0, _LANES)] = bk
        pack_v[pl.ds(_LANES, _LANES)] = plsc.bitcast(bi, jnp.float32)
        pltpu.sync_copy(pack_v, sh_p.at[pl.ds(s * _PACK, _PACK)])
        plsc.subcore_barrier()

        @pl.when(s == 0)
        def _final():
            pltpu.sync_copy(sh_p, packf)

            def final_merge(j, carry):
                bk2, bi2 = carry
                ck_s = packf[pl.ds(j * _PACK, _LANES)]
                ci_s = plsc.bitcast(packf[pl.ds(j * _PACK + _LANES, _LANES)],
                                    jnp.int32)
                return _merge_sorted(bk2, bi2, ck_s, ci_s)

            fk, fi = lax.fori_loop(0, _NTILES, final_merge, (bk0, bi0))
            idx_v[...] = fi
            pltpu.async_copy(pred_hbm.at[idx_v.at[pl.ds(0, _K)]], rows_v,
                             sem).wait()
            pltpu.sync_copy(rows_v, out_states_hbm.at[pl.ds(0, _K)])

    @pl.when((c == 1) & (s < 14))
    def _copy_bank():
        r0 = _K + s * 4
        pltpu.sync_copy(states_hbm.at[pl.ds(r0, 4)], bank_v)
        pltpu.sync_copy(bank_v, out_states_hbm.at[pl.ds(r0, 4)])

    @pl.when((c == 1) & (s == 14))
    def _strength():
        lane = lax.iota(jnp.int32, _LANES)
        pltpu.sync_copy(strength_hbm, str_v)
        s0 = str_v[pl.ds(0, _LANES)]
        str_v[pl.ds(0, _LANES)] = jnp.where(lane < _K, jnp.float32(1.0), s0)
        pltpu.sync_copy(str_v, out_strength_hbm)


def kernel(predictions, mem_states, mem_strength, top_k):
    del top_k
    pred2d = predictions.reshape(2 * _SEQ, _HID)
    norms = _tc_norms(pred2d).reshape(_SEQ)
    sc = pl.kernel(
        _sc_body,
        mesh=plsc.VectorSubcoreMesh(core_axis_name="c", subcore_axis_name="s"),
        compiler_params=pltpu.CompilerParams(needs_layout_passes=False),
        out_type=[
            jax.ShapeDtypeStruct((_SLOTS, _HID), jnp.float32),
            jax.ShapeDtypeStruct((_SLOTS,), jnp.float32),
        ],
        scratch_types=[
            pltpu.VMEM((_PER_TILE,), jnp.float32),
            pltpu.VMEM((_PACK,), jnp.float32),
            pltpu.VMEM((_NTILES * _PACK,), jnp.float32),
            pltpu.VMEM((_LANES,), jnp.int32),
            pltpu.VMEM((_K, _HID), jnp.float32),
            pltpu.VMEM((_SLOTS,), jnp.float32),
            pltpu.VMEM((4, _HID), jnp.float32),
            pltpu.VMEM_SHARED((_NTILES * _PACK,), jnp.float32),
            pltpu.SemaphoreType.DMA,
        ],
    )
    new_states, new_strength = sc(norms, pred2d, mem_states, mem_strength)
    return new_states, new_strength


# speedup vs baseline: 1.3191x; 1.0747x over previous
import jax
import jax.numpy as jnp
from jax import lax
from jax.experimental import pallas as pl
from jax.experimental.pallas import tpu as pltpu
from jax.experimental.pallas import tpu_sc as plsc

_SEQ = 4096
_HID = 4096
_SLOTS = 64
_K = 8
_LANES = 16
_NTILES = 16
_PER_TILE = _SEQ // _NTILES
_NCHUNK = _PER_TILE // _LANES
_PACK = 2 * _LANES


def _norms_body(x_ref, o_ref):
    x = x_ref[...]
    o_ref[...] = jnp.sum(x * x, axis=1)[None, None, :]


def _tc_norms(pred2d):
    nblk = 8
    rows = _SEQ // nblk
    return pl.pallas_call(
        _norms_body,
        grid=(nblk,),
        in_specs=[pl.BlockSpec((rows, _HID), lambda i: (i, 0))],
        out_specs=pl.BlockSpec((1, 1, rows), lambda i: (i, 0, 0)),
        out_shape=jax.ShapeDtypeStruct((nblk, 1, rows), jnp.float32),
    )(pred2d)


def _merge_sorted(bk, bi, ck_s, ci_s):
    ck_r = lax.rev(ck_s, (0,))
    ci_r = lax.rev(ci_s, (0,))
    keep = bk >= ck_r
    mk = jnp.where(keep, bk, ck_r)
    mi = jnp.where(keep, bi, ci_r)
    nk, ni = plsc.sort_key_val(mk, mi, descending=True)
    return nk, ni


def _sc_body(norms_hbm, pred_hbm, states_hbm, strength_hbm,
             out_states_hbm, out_strength_hbm,
             norms_v, pack_v, packf, idx_v, rows_v, str_v, bank_v,
             sh_p, sem):
    c = lax.axis_index("c")
    s = lax.axis_index("s")

    @pl.when(c == 0)
    def _topk():
        base = pl.multiple_of(s * _PER_TILE, _PER_TILE)
        pltpu.sync_copy(norms_hbm.at[pl.ds(base, _PER_TILE)], norms_v)
        lane = lax.iota(jnp.int32, _LANES)

        def local_merge(j, carry):
            bk, bi = carry
            ck = norms_v[pl.ds(j * _LANES, _LANES)]
            ci = lane + (base + j * _LANES)
            ck_s, ci_s = plsc.sort_key_val(ck, ci, descending=True)
            return _merge_sorted(bk, bi, ck_s, ci_s)

        bk0 = jnp.full((_LANES,), -jnp.inf, jnp.float32)
        bi0 = jnp.zeros((_LANES,), jnp.int32)
        bk, bi = lax.fori_loop(0, _NCHUNK, local_merge, (bk0, bi0))
        pack_v[pl.ds(0, _LANES)] = bk
        pack_v[pl.ds(_LANES, _LANES)] = plsc.bitcast(bi, jnp.float32)
        pltpu.sync_copy(pack_v, sh_p.at[pl.ds(s * _PACK, _PACK)])
        plsc.subcore_barrier()

        @pl.when(s == 0)
        def _final():
            pltpu.sync_copy(sh_p, packf)

            def final_merge(j, carry):
                bk2, bi2 = carry
                ck_s = packf[pl.ds(j * _PACK, _LANES)]
                ci_s = plsc.bitcast(packf[pl.ds(j * _PACK + _LANES, _LANES)],
                                    jnp.int32)
                return _merge_sorted(bk2, bi2, ck_s, ci_s)

            fk, fi = lax.fori_loop(0, _NTILES, final_merge, (bk0, bi0))
            idx_v[...] = fi
            pltpu.async_copy(pred_hbm.at[idx_v.at[pl.ds(0, _K)]], rows_v,
                             sem).wait()
            pltpu.sync_copy(rows_v, out_states_hbm.at[pl.ds(0, _K)])

    @pl.when((c == 1) & (s < 14))
    def _copy_bank():
        r0 = _K + s * 4
        pltpu.sync_copy(states_hbm.at[pl.ds(r0, 4)], bank_v)
        pltpu.sync_copy(bank_v, out_states_hbm.at[pl.ds(r0, 4)])

    @pl.when((c == 1) & (s == 14))
    def _strength():
        lane = lax.iota(jnp.int32, _LANES)
        pltpu.sync_copy(strength_hbm, str_v)
        s0 = str_v[pl.ds(0, _LANES)]
        str_v[pl.ds(0, _LANES)] = jnp.where(lane < _K, jnp.float32(1.0), s0)
        pltpu.sync_copy(str_v, out_strength_hbm)


def kernel(predictions, mem_states, mem_strength, top_k):
    del top_k
    pred2d = predictions.reshape(2 * _SEQ, _HID)
    norms = _tc_norms(pred2d).reshape(_SEQ)
    sc = pl.kernel(
        _sc_body,
        mesh=plsc.VectorSubcoreMesh(core_axis_name="c", subcore_axis_name="s"),
        compiler_params=pltpu.CompilerParams(needs_layout_passes=False),
        out_type=[
            jax.ShapeDtypeStruct((_SLOTS, _HID), jnp.float32),
            jax.ShapeDtypeStruct((_SLOTS,), jnp.float32),
        ],
        scratch_types=[
            pltpu.VMEM((_PER_TILE,), jnp.float32),
            pltpu.VMEM((_PACK,), jnp.float32),
            pltpu.VMEM((_NTILES * _PACK,), jnp.float32),
            pltpu.VMEM((_LANES,), jnp.int32),
            pltpu.VMEM((_K, _HID), jnp.float32),
            pltpu.VMEM((_SLOTS,), jnp.float32),
            pltpu.VMEM((4, _HID), jnp.float32),
            pltpu.VMEM_SHARED((_NTILES * _PACK,), jnp.float32),
            pltpu.SemaphoreType.DMA,
        ],
    )
    new_states, new_strength = sc(norms, pred2d, mem_states, mem_strength)
    return new_states, new_strength
